# baseline (speedup 1.0000x reference)
import functools

import jax
import jax.numpy as jnp
from jax import lax
from jax.experimental import pallas as pl
from jax.experimental.pallas import tpu as pltpu
from jax.experimental.pallas import tpu_sc as plsc

N = 10000
HID = 128
E = 320000
EPS = 1e-5

NC, NS = 2, 16
NW = NC * NS
EPW = E // NW
B = 128
CH = 80
EPWP = CH * B
EPAD = NW * EPWP - E
G = 8
NG = CH // G
NP = 10240
RPW = NP // NS

_f32 = jnp.float32

_sc_mesh = plsc.VectorSubcoreMesh(
    core_axis_name="c", subcore_axis_name="s", num_cores=NC, num_subcores=NS)


@functools.partial(
    pl.kernel,
    out_type=jax.ShapeDtypeStruct((NC, NP, HID), _f32),
    mesh=_sc_mesh,
    scratch_types=[
        pltpu.VMEM((CH, B), jnp.int32),
        pltpu.VMEM((CH, B), jnp.int32),
        pltpu.VMEM((B, HID), _f32),
        pltpu.VMEM_SHARED((NP, HID), _f32),
        pltpu.SemaphoreType.DMA,
    ],
)
def _scatter_kernel(hp_hbm, src_hbm, dst_hbm, zeros_hbm, aggp_hbm,
                    sidx_v, didx_v, rows_v, acc_sh, sem):
    c = lax.axis_index("c")
    s = lax.axis_index("s")
    r0 = s * RPW
    pltpu.sync_copy(zeros_hbm.at[pl.ds(r0, RPW)], acc_sh.at[pl.ds(r0, RPW)])
    pltpu.sync_copy(src_hbm.at[c, s], sidx_v)
    pltpu.sync_copy(dst_hbm.at[c, s], didx_v)
    plsc.subcore_barrier()

    def body(j, carry):
        pltpu.async_copy(hp_hbm.at[sidx_v.at[j]], rows_v, sem).wait()
        pltpu.sync_copy(rows_v, acc_sh.at[didx_v.at[j]], add=True)
        return carry

    lax.fori_loop(0, CH, body, 0)
    plsc.subcore_barrier()
    pltpu.sync_copy(acc_sh.at[pl.ds(r0, RPW)], aggp_hbm.at[c, pl.ds(r0, RPW)])


def _tc1_body(degp_ref, x_ref, w1_ref, b1_ref, hp_ref, dinv_ref):
    deg = degp_ref[0][:N, 0:1] + degp_ref[1][:N, 0:1] + 1.0
    dinv = jnp.broadcast_to(lax.rsqrt(deg), (N, HID))
    h = jnp.dot(x_ref[...], w1_ref[...], preferred_element_type=_f32)
    hp_ref[...] = (h + b1_ref[...]) * dinv
    dinv_ref[...] = dinv


def _tc2_body(aggp_ref, hp1_ref, dinv_ref, w2_ref, b2_ref, g1_ref, beta1_ref,
              hp2_ref):
    out1 = dinv_ref[...] * (aggp_ref[0][:N] + aggp_ref[1][:N] + hp1_ref[...])
    m = jnp.mean(out1, axis=0)
    v = jnp.mean((out1 - m) ** 2, axis=0)
    r = jnp.maximum(g1_ref[...] * (out1 - m) * lax.rsqrt(v + EPS)
                    + beta1_ref[...], 0.0)
    h2 = jnp.dot(r, w2_ref[...], preferred_element_type=_f32)
    hp2_ref[...] = (h2 + b2_ref[...]) * dinv_ref[...]


def _tc3_body(aggq_ref, hp2_ref, dinv_ref, x_ref, g2_ref, beta2_ref, out_ref):
    out2 = dinv_ref[...] * (aggq_ref[0][:N] + aggq_ref[1][:N] + hp2_ref[...])
    m = jnp.mean(out2, axis=0)
    v = jnp.mean((out2 - m) ** 2, axis=0)
    xn = g2_ref[...] * (out2 - m) * lax.rsqrt(v + EPS) + beta2_ref[...]
    out_ref[...] = jnp.maximum(xn + x_ref[...], 0.0)


_nh = jax.ShapeDtypeStruct((N, HID), _f32)
_tc1 = pl.pallas_call(_tc1_body, out_shape=(_nh, _nh))
_tc2 = pl.pallas_call(_tc2_body, out_shape=_nh)
_tc3 = pl.pallas_call(_tc3_body, out_shape=_nh)


def kernel(x, edge_index, W1, b1, g1, beta1, W2, b2, g2, beta2):
    ei = edge_index.astype(jnp.int32)
    srcp = jnp.concatenate([ei[0], jnp.zeros((EPAD,), jnp.int32)])
    pad_dst = N + (jnp.arange(EPAD, dtype=jnp.int32) % (NP - N))
    dstp = jnp.concatenate([ei[1], pad_dst])
    src3 = srcp.reshape(NC, NS, CH, B)
    dst3 = dstp.reshape(NC, NS, CH, B)
    zeros128 = jnp.zeros((NP, HID), _f32)
    ones128 = jnp.ones((NP, HID), _f32)

    degp = _scatter_kernel(ones128, src3, dst3, zeros128)
    hp1, dinv = _tc1(degp, x, W1, b1)
    aggp = _scatter_kernel(hp1, src3, dst3, zeros128)
    hp2 = _tc2(aggp, hp1, dinv, W2, b2, g1, beta1)
    aggq = _scatter_kernel(hp2, src3, dst3, zeros128)
    return _tc3(aggq, hp2, dinv, x, g2, beta2)


# speedup vs baseline: 2.9616x; 2.9616x over previous
import functools

import jax
import jax.numpy as jnp
from jax import lax
from jax.experimental import pallas as pl
from jax.experimental.pallas import tpu as pltpu
from jax.experimental.pallas import tpu_sc as plsc

N = 10000
HID = 128
E = 320000
EPS = 1e-5

NC, NS = 2, 16
NW = NC * NS
EPW = E // NW
B = 125
CH = EPW // B
NP = 10240
RPW = NP // NS

_f32 = jnp.float32

_sc_mesh = plsc.VectorSubcoreMesh(
    core_axis_name="c", subcore_axis_name="s", num_cores=NC, num_subcores=NS)


@functools.partial(
    pl.kernel,
    out_type=jax.ShapeDtypeStruct((NC, NP, HID), _f32),
    mesh=_sc_mesh,
    scratch_types=[
        pltpu.VMEM((CH, B), jnp.int32),
        pltpu.VMEM((CH, B), jnp.int32),
        pltpu.VMEM((B, HID), _f32),
        pltpu.VMEM_SHARED((NP, HID), _f32),
        pltpu.SemaphoreType.DMA,
    ],
)
def _scatter_kernel(hp_hbm, src_hbm, dst_hbm, zeros_hbm, aggp_hbm,
                    sidx_v, didx_v, rows_v, acc_sh, sem):
    c = lax.axis_index("c")
    s = lax.axis_index("s")
    r0 = s * RPW
    pltpu.sync_copy(zeros_hbm.at[pl.ds(r0, RPW)], acc_sh.at[pl.ds(r0, RPW)])
    pltpu.sync_copy(src_hbm.at[c, s], sidx_v)
    pltpu.sync_copy(dst_hbm.at[c, s], didx_v)
    plsc.subcore_barrier()

    def body(j, carry):
        pltpu.async_copy(hp_hbm.at[sidx_v.at[j]], rows_v, sem).wait()
        pltpu.sync_copy(rows_v, acc_sh.at[didx_v.at[j]], add=True)
        return carry

    lax.fori_loop(0, CH, body, 0)
    plsc.subcore_barrier()
    pltpu.sync_copy(acc_sh.at[pl.ds(r0, RPW)], aggp_hbm.at[c, pl.ds(r0, RPW)])


def _tc1_body(degp_ref, x_ref, w1_ref, b1_ref, hp_ref, dinv_ref):
    deg = degp_ref[0][:N, 0:1] + degp_ref[1][:N, 0:1] + 1.0
    dinv = jnp.broadcast_to(lax.rsqrt(deg), (N, HID))
    h = jnp.dot(x_ref[...], w1_ref[...], preferred_element_type=_f32)
    hp_ref[...] = (h + b1_ref[...]) * dinv
    dinv_ref[...] = dinv


def _tc2_body(aggp_ref, hp1_ref, dinv_ref, w2_ref, b2_ref, g1_ref, beta1_ref,
              hp2_ref):
    out1 = dinv_ref[...] * (aggp_ref[0][:N] + aggp_ref[1][:N] + hp1_ref[...])
    m = jnp.mean(out1, axis=0)
    v = jnp.mean((out1 - m) ** 2, axis=0)
    r = jnp.maximum(g1_ref[...] * (out1 - m) * lax.rsqrt(v + EPS)
                    + beta1_ref[...], 0.0)
    h2 = jnp.dot(r, w2_ref[...], preferred_element_type=_f32)
    hp2_ref[...] = (h2 + b2_ref[...]) * dinv_ref[...]


def _tc3_body(aggq_ref, hp2_ref, dinv_ref, x_ref, g2_ref, beta2_ref, out_ref):
    out2 = dinv_ref[...] * (aggq_ref[0][:N] + aggq_ref[1][:N] + hp2_ref[...])
    m = jnp.mean(out2, axis=0)
    v = jnp.mean((out2 - m) ** 2, axis=0)
    xn = g2_ref[...] * (out2 - m) * lax.rsqrt(v + EPS) + beta2_ref[...]
    out_ref[...] = jnp.maximum(xn + x_ref[...], 0.0)


_nh = jax.ShapeDtypeStruct((N, HID), _f32)
_tc1 = pl.pallas_call(_tc1_body, out_shape=(_nh, _nh))
_tc2 = pl.pallas_call(_tc2_body, out_shape=_nh)
_tc3 = pl.pallas_call(_tc3_body, out_shape=_nh)


def kernel(x, edge_index, W1, b1, g1, beta1, W2, b2, g2, beta2):
    ei = edge_index.astype(jnp.int32)
    src3 = ei[0].reshape(NC, NS, CH, B)
    dst3 = ei[1].reshape(NC, NS, CH, B)
    zeros128 = jnp.zeros((NP, HID), _f32)
    ones128 = jnp.ones((NP, HID), _f32)

    degp = _scatter_kernel(ones128, src3, dst3, zeros128)
    hp1, dinv = _tc1(degp, x, W1, b1)
    aggp = _scatter_kernel(hp1, src3, dst3, zeros128)
    hp2 = _tc2(aggp, hp1, dinv, W2, b2, g1, beta1)
    aggq = _scatter_kernel(hp2, src3, dst3, zeros128)
    return _tc3(aggq, hp2, dinv, x, g2, beta2)


# speedup vs baseline: 4.4758x; 1.5113x over previous
import functools

import jax
import jax.numpy as jnp
from jax import lax
from jax.experimental import pallas as pl
from jax.experimental.pallas import tpu as pltpu
from jax.experimental.pallas import tpu_sc as plsc

N = 10000
HID = 128
E = 320000
EPS = 1e-5

NC, NS = 2, 16
NW = NC * NS
EPW = E // NW
B = 125
CH = EPW // B
NP = 10240
RPW = NP // NS

_f32 = jnp.float32

_sc_mesh = plsc.VectorSubcoreMesh(
    core_axis_name="c", subcore_axis_name="s", num_cores=NC, num_subcores=NS)


@functools.partial(
    pl.kernel,
    out_type=jax.ShapeDtypeStruct((NC, NP, HID), _f32),
    mesh=_sc_mesh,
    scratch_types=[
        pltpu.VMEM((CH, B), jnp.int32),
        pltpu.VMEM((CH // 2, B), jnp.int32),
        pltpu.VMEM((B, HID), _f32),
        pltpu.VMEM((B, HID), _f32),
        pltpu.VMEM_SHARED((NP, HID), _f32),
        pltpu.SemaphoreType.DMA,
        pltpu.SemaphoreType.DMA,
    ],
)
def _scatter_kernel(hp_hbm, src_hbm, dst_hbm, zeros_hbm, aggp_hbm,
                    sidx_v, didx_v, rows0_v, rows1_v, acc_sh, sem0, sem1):
    c = lax.axis_index("c")
    s = lax.axis_index("s")
    r0 = s * RPW
    hch = CH // 2
    pltpu.sync_copy(zeros_hbm.at[pl.ds(r0, RPW)], acc_sh.at[pl.ds(r0, RPW)])
    pltpu.sync_copy(src_hbm.at[c, s], sidx_v)
    pltpu.sync_copy(dst_hbm.at[c, s, pl.ds(0, hch)], didx_v)
    plsc.subcore_barrier()

    pltpu.async_copy(hp_hbm.at[sidx_v.at[0]], rows0_v, sem0)
    for half in range(2):
        base = half * hch

        @pl.loop(base, base + hch, step=2)
        def _chunks(j):
            pltpu.async_copy(hp_hbm.at[sidx_v.at[j + 1]], rows1_v, sem1)
            pltpu.make_async_copy(hp_hbm.at[sidx_v.at[j]], rows0_v, sem0).wait()
            pltpu.sync_copy(rows0_v, acc_sh.at[didx_v.at[j - base]], add=True)
            jn = lax.min(j + 2, CH - 1)
            pltpu.async_copy(hp_hbm.at[sidx_v.at[jn]], rows0_v, sem0)
            pltpu.make_async_copy(hp_hbm.at[sidx_v.at[j + 1]], rows1_v, sem1).wait()
            pltpu.sync_copy(rows1_v, acc_sh.at[didx_v.at[j + 1 - base]], add=True)

        if half == 0:
            pltpu.sync_copy(dst_hbm.at[c, s, pl.ds(hch, hch)], didx_v)

    pltpu.make_async_copy(hp_hbm.at[sidx_v.at[CH - 1]], rows0_v, sem0).wait()
    plsc.subcore_barrier()
    pltpu.sync_copy(acc_sh.at[pl.ds(r0, RPW)], aggp_hbm.at[c, pl.ds(r0, RPW)])


def _tc1_body(degp_ref, x_ref, w1_ref, b1_ref, hp_ref, dinv_ref):
    deg = degp_ref[0][:N, 0:1] + degp_ref[1][:N, 0:1] + 1.0
    dinv = jnp.broadcast_to(lax.rsqrt(deg), (N, HID))
    h = jnp.dot(x_ref[...], w1_ref[...], preferred_element_type=_f32)
    hp_ref[...] = (h + b1_ref[...]) * dinv
    dinv_ref[...] = dinv


def _tc2_body(aggp_ref, hp1_ref, dinv_ref, w2_ref, b2_ref, g1_ref, beta1_ref,
              hp2_ref):
    out1 = dinv_ref[...] * (aggp_ref[0][:N] + aggp_ref[1][:N] + hp1_ref[...])
    m = jnp.mean(out1, axis=0)
    v = jnp.mean((out1 - m) ** 2, axis=0)
    r = jnp.maximum(g1_ref[...] * (out1 - m) * lax.rsqrt(v + EPS)
                    + beta1_ref[...], 0.0)
    h2 = jnp.dot(r, w2_ref[...], preferred_element_type=_f32)
    hp2_ref[...] = (h2 + b2_ref[...]) * dinv_ref[...]


def _tc3_body(aggq_ref, hp2_ref, dinv_ref, x_ref, g2_ref, beta2_ref, out_ref):
    out2 = dinv_ref[...] * (aggq_ref[0][:N] + aggq_ref[1][:N] + hp2_ref[...])
    m = jnp.mean(out2, axis=0)
    v = jnp.mean((out2 - m) ** 2, axis=0)
    xn = g2_ref[...] * (out2 - m) * lax.rsqrt(v + EPS) + beta2_ref[...]
    out_ref[...] = jnp.maximum(xn + x_ref[...], 0.0)


_nh = jax.ShapeDtypeStruct((N, HID), _f32)
_tc1 = pl.pallas_call(_tc1_body, out_shape=(_nh, _nh))
_tc2 = pl.pallas_call(_tc2_body, out_shape=_nh)
_tc3 = pl.pallas_call(_tc3_body, out_shape=_nh)


def kernel(x, edge_index, W1, b1, g1, beta1, W2, b2, g2, beta2):
    ei = edge_index.astype(jnp.int32)
    src3 = ei[0].reshape(NC, NS, CH, B)
    dst3 = ei[1].reshape(NC, NS, CH, B)
    zeros128 = jnp.zeros((NP, HID), _f32)
    ones128 = jnp.ones((NP, HID), _f32)

    degp = _scatter_kernel(ones128, src3, dst3, zeros128)
    hp1, dinv = _tc1(degp, x, W1, b1)
    aggp = _scatter_kernel(hp1, src3, dst3, zeros128)
    hp2 = _tc2(aggp, hp1, dinv, W2, b2, g1, beta1)
    aggq = _scatter_kernel(hp2, src3, dst3, zeros128)
    return _tc3(aggq, hp2, dinv, x, g2, beta2)


# speedup vs baseline: 4.9773x; 1.1120x over previous
import functools

import jax
import jax.numpy as jnp
from jax import lax
from jax.experimental import pallas as pl
from jax.experimental.pallas import tpu as pltpu
from jax.experimental.pallas import tpu_sc as plsc

N = 10000
HID = 128
E = 320000
EPS = 1e-5

NC, NS = 2, 16
NW = NC * NS
EPW = E // NW
B = 125
CH = EPW // B
BC = 80
CHC = EPW // BC
NP = 10240
RPW = NP // NS

_f32 = jnp.float32

_sc_mesh = plsc.VectorSubcoreMesh(
    core_axis_name="c", subcore_axis_name="s", num_cores=NC, num_subcores=NS)


@functools.partial(
    pl.kernel,
    out_type=jax.ShapeDtypeStruct((NC, NP, HID), _f32),
    mesh=_sc_mesh,
    scratch_types=[
        pltpu.VMEM((CHC, BC), jnp.int32),
        pltpu.VMEM((BC, HID), _f32),
        pltpu.VMEM_SHARED((NP, HID), _f32),
        pltpu.SemaphoreType.DMA,
    ],
)
def _count_kernel(dst_hbm, ones_hbm, zeros_hbm, degp_hbm,
                  didx_v, pay_v, acc_sh, sem):
    c = lax.axis_index("c")
    s = lax.axis_index("s")
    r0 = s * RPW
    pltpu.sync_copy(zeros_hbm.at[pl.ds(r0, RPW)], acc_sh.at[pl.ds(r0, RPW)])
    pltpu.sync_copy(dst_hbm.at[c, s], didx_v)
    pltpu.sync_copy(ones_hbm.at[pl.ds(0, BC)], pay_v)
    plsc.subcore_barrier()

    pltpu.async_copy(pay_v, acc_sh.at[didx_v.at[0]], sem, add=True)

    @pl.loop(1, CHC)
    def _chunks(j):
        pltpu.async_copy(pay_v, acc_sh.at[didx_v.at[j]], sem, add=True)
        pltpu.make_async_copy(pay_v, acc_sh.at[didx_v.at[0]], sem).wait()

    pltpu.make_async_copy(pay_v, acc_sh.at[didx_v.at[0]], sem).wait()
    plsc.subcore_barrier()
    pltpu.sync_copy(acc_sh.at[pl.ds(r0, RPW)], degp_hbm.at[c, pl.ds(r0, RPW)])


@functools.partial(
    pl.kernel,
    out_type=jax.ShapeDtypeStruct((NC, NP, HID), _f32),
    mesh=_sc_mesh,
    scratch_types=[
        pltpu.VMEM((CH, B), jnp.int32),
        pltpu.VMEM((CH // 2, B), jnp.int32),
        pltpu.VMEM((B, HID), _f32),
        pltpu.VMEM((B, HID), _f32),
        pltpu.VMEM_SHARED((NP, HID), _f32),
        pltpu.SemaphoreType.DMA,
        pltpu.SemaphoreType.DMA,
    ],
)
def _scatter_kernel(hp_hbm, src_hbm, dst_hbm, zeros_hbm, aggp_hbm,
                    sidx_v, didx_v, rows0_v, rows1_v, acc_sh, sem0, sem1):
    c = lax.axis_index("c")
    s = lax.axis_index("s")
    r0 = s * RPW
    hch = CH // 2
    pltpu.sync_copy(zeros_hbm.at[pl.ds(r0, RPW)], acc_sh.at[pl.ds(r0, RPW)])
    pltpu.sync_copy(src_hbm.at[c, s], sidx_v)
    pltpu.sync_copy(dst_hbm.at[c, s, pl.ds(0, hch)], didx_v)
    plsc.subcore_barrier()

    pltpu.async_copy(hp_hbm.at[sidx_v.at[0]], rows0_v, sem0)
    for half in range(2):
        base = half * hch

        @pl.loop(base, base + hch, step=2)
        def _chunks(j):
            pltpu.async_copy(hp_hbm.at[sidx_v.at[j + 1]], rows1_v, sem1)
            pltpu.make_async_copy(hp_hbm.at[sidx_v.at[j]], rows0_v, sem0).wait()
            pltpu.sync_copy(rows0_v, acc_sh.at[didx_v.at[j - base]], add=True)
            jn = lax.min(j + 2, CH - 1)
            pltpu.async_copy(hp_hbm.at[sidx_v.at[jn]], rows0_v, sem0)
            pltpu.make_async_copy(hp_hbm.at[sidx_v.at[j + 1]], rows1_v, sem1).wait()
            pltpu.sync_copy(rows1_v, acc_sh.at[didx_v.at[j + 1 - base]], add=True)

        if half == 0:
            pltpu.sync_copy(dst_hbm.at[c, s, pl.ds(hch, hch)], didx_v)

    pltpu.make_async_copy(hp_hbm.at[sidx_v.at[CH - 1]], rows0_v, sem0).wait()
    plsc.subcore_barrier()
    pltpu.sync_copy(acc_sh.at[pl.ds(r0, RPW)], aggp_hbm.at[c, pl.ds(r0, RPW)])


def _tc1_body(degp_ref, x_ref, w1_ref, b1_ref, hp_ref, dinv_ref):
    deg = degp_ref[0][:N, 0:1] + degp_ref[1][:N, 0:1] + 1.0
    dinv = jnp.broadcast_to(lax.rsqrt(deg), (N, HID))
    h = jnp.dot(x_ref[...], w1_ref[...], preferred_element_type=_f32)
    hp_ref[...] = (h + b1_ref[...]) * dinv
    dinv_ref[...] = dinv


def _tc2_body(aggp_ref, hp1_ref, dinv_ref, w2_ref, b2_ref, g1_ref, beta1_ref,
              hp2_ref):
    out1 = dinv_ref[...] * (aggp_ref[0][:N] + aggp_ref[1][:N] + hp1_ref[...])
    m = jnp.mean(out1, axis=0)
    v = jnp.mean((out1 - m) ** 2, axis=0)
    r = jnp.maximum(g1_ref[...] * (out1 - m) * lax.rsqrt(v + EPS)
                    + beta1_ref[...], 0.0)
    h2 = jnp.dot(r, w2_ref[...], preferred_element_type=_f32)
    hp2_ref[...] = (h2 + b2_ref[...]) * dinv_ref[...]


def _tc3_body(aggq_ref, hp2_ref, dinv_ref, x_ref, g2_ref, beta2_ref, out_ref):
    out2 = dinv_ref[...] * (aggq_ref[0][:N] + aggq_ref[1][:N] + hp2_ref[...])
    m = jnp.mean(out2, axis=0)
    v = jnp.mean((out2 - m) ** 2, axis=0)
    xn = g2_ref[...] * (out2 - m) * lax.rsqrt(v + EPS) + beta2_ref[...]
    out_ref[...] = jnp.maximum(xn + x_ref[...], 0.0)


_nh = jax.ShapeDtypeStruct((N, HID), _f32)
_tc1 = pl.pallas_call(_tc1_body, out_shape=(_nh, _nh))
_tc2 = pl.pallas_call(_tc2_body, out_shape=_nh)
_tc3 = pl.pallas_call(_tc3_body, out_shape=_nh)


def kernel(x, edge_index, W1, b1, g1, beta1, W2, b2, g2, beta2):
    ei = edge_index.astype(jnp.int32)
    src3 = ei[0].reshape(NC, NS, CH, B)
    dst3 = ei[1].reshape(NC, NS, CH, B)
    dst3c = ei[1].reshape(NC, NS, CHC, BC)
    zeros128 = jnp.zeros((NP, HID), _f32)
    ones128 = jnp.ones((NP, HID), _f32)

    degp = _count_kernel(dst3c, ones128, zeros128)
    hp1, dinv = _tc1(degp, x, W1, b1)
    aggp = _scatter_kernel(hp1, src3, dst3, zeros128)
    hp2 = _tc2(aggp, hp1, dinv, W2, b2, g1, beta1)
    aggq = _scatter_kernel(hp2, src3, dst3, zeros128)
    return _tc3(aggq, hp2, dinv, x, g2, beta2)
